# Initial kernel scaffold; baseline (speedup 1.0000x reference)
import jax
import jax.numpy as jnp
from jax.experimental import pallas as pl


def kernel(x, edge_index, batch, Wl1, bl1, Wr1, br1, att1, b1, Wl2, bl2, Wr2, br2, att2, b2, Wl3, bl3, Wr3, br3, att3, b3, Wlin, blin):
    raise NotImplementedError("write your pallas kernel here")


# speedup vs baseline: 1.8014x; 1.8014x over previous
import functools

import jax
import jax.numpy as jnp
from jax.experimental import pallas as pl

N = 10000
G = 64


def _layer(x, src, dst, Wl, bl, Wr, br, att, bias):
    xl = x @ Wl + bl
    xr = x @ Wr + br
    e = jax.nn.leaky_relu(xl[src] + xr[dst], negative_slope=0.2)
    a = jnp.exp(e @ att)
    num = jax.ops.segment_sum(a[:, None] * xl[src], dst, num_segments=N)
    den = jax.ops.segment_sum(a, dst, num_segments=N)
    return num / den[:, None] + bias


def _final_matmul_kernel(p_ref, w_ref, b_ref, o_ref):
    o_ref[...] = p_ref[...] @ w_ref[...] + b_ref[...]


def kernel(x, edge_index, batch,
           Wl1, bl1, Wr1, br1, att1, b1,
           Wl2, bl2, Wr2, br2, att2, b2,
           Wl3, bl3, Wr3, br3, att3, b3,
           Wlin, blin):
    loops = jnp.arange(N, dtype=edge_index.dtype)
    src = jnp.concatenate([edge_index[0], loops])
    dst = jnp.concatenate([edge_index[1], loops])
    h = _layer(x, src, dst, Wl1, bl1, Wr1, br1, att1, b1)
    h = jax.nn.gelu(h, approximate=False)
    h = _layer(h, src, dst, Wl2, bl2, Wr2, br2, att2, b2)
    h = jax.nn.gelu(h, approximate=False)
    h = _layer(h, src, dst, Wl3, bl3, Wr3, br3, att3, b3)
    sums = jax.ops.segment_sum(h, batch, num_segments=G)
    cnt = jax.ops.segment_sum(jnp.ones((N,), h.dtype), batch, num_segments=G)
    pooled = sums / jnp.maximum(cnt, 1.0)[:, None]
    out = pl.pallas_call(
        _final_matmul_kernel,
        out_shape=jax.ShapeDtypeStruct((G, Wlin.shape[1]), jnp.float32),
    )(pooled, Wlin, blin[None, :])
    return out


# speedup vs baseline: 4.5586x; 2.5305x over previous
import functools

import jax
import jax.numpy as jnp
from jax import lax
from jax.experimental import pallas as pl
from jax.experimental.pallas import tpu as pltpu
from jax.experimental.pallas import tpu_sc as plsc

N = 10000
G = 64
FT_IN = 128
HID = 256
NP = 10240
E0 = 330000
W1 = 32
C1 = 64
EW1 = 10368
NC1 = EW1 // C1
E_PAD = EW1 * W1
C2 = 64
EW2 = E_PAD // 16
NC2 = EW2 // C2
RPW = NP // 16
DW = 16

_SC_MESH = dict(core_axis_name="c", subcore_axis_name="s")

_GDN = lax.GatherDimensionNumbers(
    offset_dims=(), collapsed_slice_dims=(0,), start_index_map=(0,))


def _vshuf(v, idx):
    return lax.gather(v, idx[:, None], _GDN, (1,),
                      mode=lax.GatherScatterMode.PROMISE_IN_BOUNDS)


def _lanesum(v, lane):
    for sh in (8, 4, 2, 1):
        v = v + _vshuf(v, lane ^ sh)
    return v


def _proj_body(x_ref, wl_ref, bl_ref, wr_ref, br_ref,
               xlf_ref, xrf_ref, xlc_ref):
    xb = x_ref[...]
    l = jnp.dot(xb, wl_ref[...], preferred_element_type=jnp.float32) + bl_ref[...]
    r = jnp.dot(xb, wr_ref[...], preferred_element_type=jnp.float32) + br_ref[...]
    xlf_ref[...] = l
    xrf_ref[...] = r
    xlc_ref[0] = l[:, :FT_IN]
    xlc_ref[1] = l[:, FT_IN:]


def _proj(x, wl, bl, wr, br, din):
    grid = NP // 1024
    return pl.pallas_call(
        _proj_body,
        grid=(grid,),
        in_specs=[
            pl.BlockSpec((1024, din), lambda i: (i, 0)),
            pl.BlockSpec((din, HID), lambda i: (0, 0)),
            pl.BlockSpec((1, HID), lambda i: (0, 0)),
            pl.BlockSpec((din, HID), lambda i: (0, 0)),
            pl.BlockSpec((1, HID), lambda i: (0, 0)),
        ],
        out_specs=[
            pl.BlockSpec((1024, HID), lambda i: (i, 0)),
            pl.BlockSpec((1024, HID), lambda i: (i, 0)),
            pl.BlockSpec((2, 1024, FT_IN), lambda i: (0, i, 0)),
        ],
        out_shape=[
            jax.ShapeDtypeStruct((NP, HID), jnp.float32),
            jax.ShapeDtypeStruct((NP, HID), jnp.float32),
            jax.ShapeDtypeStruct((2, NP, FT_IN), jnp.float32),
        ],
    )(x, wl, bl[None, :], wr, br[None, :])


def _epi_body(num_ref, den_ref, b_ref, wl_ref, bl_ref, wr_ref, br_ref,
              xlf_ref, xrf_ref, xlc_ref):
    d = jnp.maximum(den_ref[...][:, 0:1], 1e-30)
    h = jnp.concatenate([num_ref[0] / d, num_ref[1] / d], axis=1) + b_ref[...]
    h = h * 0.5 * (1.0 + lax.erf(h * 0.7071067811865476))
    l = jnp.dot(h, wl_ref[...], preferred_element_type=jnp.float32) + bl_ref[...]
    r = jnp.dot(h, wr_ref[...], preferred_element_type=jnp.float32) + br_ref[...]
    xlf_ref[...] = l
    xrf_ref[...] = r
    xlc_ref[0] = l[:, :FT_IN]
    xlc_ref[1] = l[:, FT_IN:]


def _epilogue_proj(num, den, b, wl, bl, wr, br):
    grid = NP // 1024
    return pl.pallas_call(
        _epi_body,
        grid=(grid,),
        in_specs=[
            pl.BlockSpec((2, 1024, FT_IN), lambda i: (0, i, 0)),
            pl.BlockSpec((1024, DW), lambda i: (i, 0)),
            pl.BlockSpec((1, HID), lambda i: (0, 0)),
            pl.BlockSpec((HID, HID), lambda i: (0, 0)),
            pl.BlockSpec((1, HID), lambda i: (0, 0)),
            pl.BlockSpec((HID, HID), lambda i: (0, 0)),
            pl.BlockSpec((1, HID), lambda i: (0, 0)),
        ],
        out_specs=[
            pl.BlockSpec((1024, HID), lambda i: (i, 0)),
            pl.BlockSpec((1024, HID), lambda i: (i, 0)),
            pl.BlockSpec((2, 1024, FT_IN), lambda i: (0, i, 0)),
        ],
        out_shape=[
            jax.ShapeDtypeStruct((NP, HID), jnp.float32),
            jax.ShapeDtypeStruct((NP, HID), jnp.float32),
            jax.ShapeDtypeStruct((2, NP, FT_IN), jnp.float32),
        ],
    )(num, den, b[None, :], wl, bl[None, :], wr, br[None, :])


def _pool_body(num_ref, den_ref, b_ref, batch_ref, wlin_ref, blin_ref, out_ref):
    d = jnp.maximum(den_ref[...][:, 0:1], 1e-30)
    h = jnp.concatenate([num_ref[0] / d, num_ref[1] / d], axis=1) + b_ref[...]
    gid = lax.broadcasted_iota(jnp.int32, (G, NP), 0)
    m = (gid == batch_ref[...]).astype(jnp.float32)
    cnt = jnp.sum(m, axis=1, keepdims=True)
    sums = jnp.dot(m, h, preferred_element_type=jnp.float32)
    pooled = sums / jnp.maximum(cnt, 1.0)
    out_ref[...] = (jnp.dot(pooled, wlin_ref[...],
                            preferred_element_type=jnp.float32) + blin_ref[...])


def _pool(num, den, b, batch_pad, wlin, blin):
    return pl.pallas_call(
        _pool_body,
        out_shape=jax.ShapeDtypeStruct((G, FT_IN), jnp.float32),
    )(num, den, b[None, :], batch_pad, wlin, blin[None, :])


def _phase1_body(srcp_hbm, dstp_hbm, att_hbm, xlf_hbm, xrf_hbm,
                 a_hbm, den_hbm,
                 sidx, didx, lrow, rrow, attv, av, denbuf, den_sp, sem):
    c = lax.axis_index("c")
    s = lax.axis_index("s")
    wid = s * 2 + c
    pltpu.sync_copy(att_hbm, attv)
    atts = [attv[k] for k in range(16)]
    lane = lax.iota(jnp.int32, 16)
    col0 = lane * 0
    zero16 = col0.astype(jnp.float32)
    for i in range(C1):
        denbuf[i] = zero16
    for j in range(RPW // C1):
        pltpu.sync_copy(denbuf, den_sp.at[pl.ds(s * RPW + j * C1, C1)])
    plsc.subcore_barrier()

    def chunk(ci, carry):
        off = wid * EW1 + ci * C1
        pltpu.sync_copy(srcp_hbm.at[0, pl.ds(off, C1)], sidx)
        pltpu.sync_copy(dstp_hbm.at[0, pl.ds(off, C1)], didx)
        pltpu.async_copy(xlf_hbm.at[sidx], lrow, sem).wait()
        pltpu.async_copy(xrf_hbm.at[didx], rrow, sem).wait()

        def group(g, carry2):
            def edge(e, res):
                row = g * 16 + e
                acc = zero16
                for k in range(16):
                    z = lrow[row, pl.ds(k * 16, 16)] + rrow[row, pl.ds(k * 16, 16)]
                    z = jnp.maximum(z, z * 0.2)
                    acc = acc + atts[k] * z
                return jnp.where(lane == e, _lanesum(acc, lane), res)

            res = lax.fori_loop(0, 16, edge, zero16)
            a16 = jnp.exp(res)
            av[pl.ds(g * 16, 16)] = a16
            for e in range(16):
                sc = _vshuf(a16, col0 + e)
                denbuf[g * 16 + e] = jnp.where(lane == 0, sc, zero16)
            return carry2

        lax.fori_loop(0, C1 // 16, group, 0)
        pltpu.sync_copy(av, a_hbm.at[pl.ds(off, C1)])
        pltpu.sync_copy(denbuf, den_sp.at[didx], add=True)
        return carry

    lax.fori_loop(0, NC1, chunk, 0)
    plsc.subcore_barrier()
    for j in range(RPW // C1):
        sl = pl.ds(s * RPW + j * C1, C1)
        pltpu.sync_copy(den_sp.at[sl], denbuf)
        pltpu.sync_copy(denbuf, den_hbm.at[pl.ds(c * NP + s * RPW + j * C1, C1)])


def _phase1(srcp2, dstp2, attm, xlf, xrf):
    mesh = plsc.VectorSubcoreMesh(**_SC_MESH)
    f = functools.partial(
        pl.kernel,
        mesh=mesh,
        out_type=[
            jax.ShapeDtypeStruct((E_PAD,), jnp.float32),
            jax.ShapeDtypeStruct((2 * NP, DW), jnp.float32),
        ],
        scratch_types=[
            pltpu.VMEM((C1,), jnp.int32),
            pltpu.VMEM((C1,), jnp.int32),
            pltpu.VMEM((C1, HID), jnp.float32),
            pltpu.VMEM((C1, HID), jnp.float32),
            pltpu.VMEM((16, 16), jnp.float32),
            pltpu.VMEM((C1,), jnp.float32),
            pltpu.VMEM((C1, DW), jnp.float32),
            pltpu.VMEM_SHARED((NP, DW), jnp.float32),
            pltpu.SemaphoreType.DMA,
        ],
    )(_phase1_body)
    return f(srcp2, dstp2, attm, xlf, xrf)


def _phase2_body(srcp_hbm, dstp_hbm, a_hbm, xlc_hbm,
                 num_hbm,
                 sidx, didx, av, rows, acc_sp, sem):
    c = lax.axis_index("c")
    s = lax.axis_index("s")
    lane = lax.iota(jnp.int32, 16)
    col0 = lane * 0
    zero16 = col0.astype(jnp.float32)
    for i in range(C2):
        for k in range(FT_IN // 16):
            rows[i, pl.ds(k * 16, 16)] = zero16
    for j in range(RPW // C2):
        pltpu.sync_copy(rows, acc_sp.at[pl.ds(s * RPW + j * C2, C2)])
    plsc.subcore_barrier()

    def chunk(ci, carry):
        off = s * EW2 + ci * C2
        pltpu.sync_copy(srcp_hbm.at[pl.ds(c * E_PAD + off, C2)], sidx)
        pltpu.sync_copy(dstp_hbm.at[0, pl.ds(off, C2)], didx)
        pltpu.sync_copy(a_hbm.at[pl.ds(off, C2)], av)
        pltpu.async_copy(xlc_hbm.at[sidx], rows, sem).wait()
        for g in range(C2 // 16):
            a16 = av[pl.ds(g * 16, 16)]
            for e in range(16):
                row = g * 16 + e
                sc = _vshuf(a16, col0 + e)
                for k in range(FT_IN // 16):
                    sl = pl.ds(k * 16, 16)
                    rows[row, sl] = rows[row, sl] * sc
        pltpu.sync_copy(rows, acc_sp.at[didx], add=True)
        return carry

    lax.fori_loop(0, NC2, chunk, 0)
    plsc.subcore_barrier()
    for j in range(RPW // C2):
        sl = pl.ds(s * RPW + j * C2, C2)
        pltpu.sync_copy(acc_sp.at[sl], rows)
        pltpu.sync_copy(rows, num_hbm.at[pl.ds(c * NP + s * RPW + j * C2, C2)])


def _phase2(srcp2, dstp2, a, xlc):
    mesh = plsc.VectorSubcoreMesh(**_SC_MESH)
    srcp_flat = srcp2.reshape(2 * E_PAD)
    f = functools.partial(
        pl.kernel,
        mesh=mesh,
        out_type=jax.ShapeDtypeStruct((2 * NP, FT_IN), jnp.float32),
        scratch_types=[
            pltpu.VMEM((C2,), jnp.int32),
            pltpu.VMEM((C2,), jnp.int32),
            pltpu.VMEM((C2,), jnp.float32),
            pltpu.VMEM((C2, FT_IN), jnp.float32),
            pltpu.VMEM_SHARED((NP, FT_IN), jnp.float32),
            pltpu.SemaphoreType.DMA,
        ],
    )(_phase2_body)
    return f(srcp_flat, dstp2, a, xlc)


def kernel(x, edge_index, batch,
           Wl1, bl1, Wr1, br1, att1, b1,
           Wl2, bl2, Wr2, br2, att2, b2,
           Wl3, bl3, Wr3, br3, att3, b3,
           Wlin, blin):
    i32 = jnp.int32
    loops = jnp.arange(N, dtype=i32)
    pad_e = jnp.full((E_PAD - E0,), N, dtype=i32)
    srcp = jnp.concatenate([edge_index[0].astype(i32), loops, pad_e])
    dstp = jnp.concatenate([edge_index[1].astype(i32), loops, pad_e])
    srcp2 = jnp.stack([srcp, srcp + NP])
    dstp2 = jnp.stack([dstp, dstp + NP])
    xpad = jnp.zeros((NP, FT_IN), jnp.float32).at[:N].set(x)
    batch_pad = jnp.full((1, NP), G, dtype=i32).at[0, :N].set(batch.astype(i32))

    params = [
        (Wl1, bl1, Wr1, br1, att1, b1),
        (Wl2, bl2, Wr2, br2, att2, b2),
        (Wl3, bl3, Wr3, br3, att3, b3),
    ]

    xlf, xrf, xlc = _proj(xpad, Wl1, bl1, Wr1, br1, FT_IN)
    for li in range(3):
        att = params[li][4].reshape(16, 16)
        a, den2 = _phase1(srcp2, dstp2, att, xlf, xrf)
        den = den2[:NP] + den2[NP:]
        xlc2 = xlc.reshape(2 * NP, FT_IN)
        numf = _phase2(srcp2, dstp2, a, xlc2)
        num = numf.reshape(2, NP, FT_IN)
        b = params[li][5]
        if li < 2:
            wl, bl, wr, br = params[li + 1][:4]
            xlf, xrf, xlc = _epilogue_proj(num, den, b, wl, bl, wr, br)
        else:
            out = _pool(num, den, b, batch_pad, Wlin, blin)
    return out


# speedup vs baseline: 4.9548x; 1.0869x over previous
import functools

import jax
import jax.numpy as jnp
from jax import lax
from jax.experimental import pallas as pl
from jax.experimental.pallas import tpu as pltpu
from jax.experimental.pallas import tpu_sc as plsc

N = 10000
G = 64
FT_IN = 128
HID = 256
NP = 10240
E0 = 330000
W1 = 32
C1 = 64
EW1 = 10368
NC1 = EW1 // C1
E_PAD = EW1 * W1
C2 = 64
EW2 = E_PAD // 16
NC2 = EW2 // C2
RPW = NP // 16
DW = 16

_SC_MESH = dict(core_axis_name="c", subcore_axis_name="s")
_DB2 = False

_GDN = lax.GatherDimensionNumbers(
    offset_dims=(), collapsed_slice_dims=(0,), start_index_map=(0,))


def _vshuf(v, idx):
    return lax.gather(v, idx[:, None], _GDN, (1,),
                      mode=lax.GatherScatterMode.PROMISE_IN_BOUNDS)


def _lanesum(v, lane):
    for sh in (8, 4, 2, 1):
        v = v + _vshuf(v, lane ^ sh)
    return v


def _proj_body(x_ref, wl_ref, bl_ref, wr_ref, br_ref,
               xlf_ref, xrf_ref, xlc_ref):
    xb = x_ref[...]
    l = jnp.dot(xb, wl_ref[...], preferred_element_type=jnp.float32) + bl_ref[...]
    r = jnp.dot(xb, wr_ref[...], preferred_element_type=jnp.float32) + br_ref[...]
    xlf_ref[...] = l
    xrf_ref[...] = r
    xlc_ref[0] = l[:, :FT_IN]
    xlc_ref[1] = l[:, FT_IN:]


def _proj(x, wl, bl, wr, br, din):
    grid = NP // 1024
    return pl.pallas_call(
        _proj_body,
        grid=(grid,),
        in_specs=[
            pl.BlockSpec((1024, din), lambda i: (i, 0)),
            pl.BlockSpec((din, HID), lambda i: (0, 0)),
            pl.BlockSpec((1, HID), lambda i: (0, 0)),
            pl.BlockSpec((din, HID), lambda i: (0, 0)),
            pl.BlockSpec((1, HID), lambda i: (0, 0)),
        ],
        out_specs=[
            pl.BlockSpec((1024, HID), lambda i: (i, 0)),
            pl.BlockSpec((1024, HID), lambda i: (i, 0)),
            pl.BlockSpec((2, 1024, FT_IN), lambda i: (0, i, 0)),
        ],
        out_shape=[
            jax.ShapeDtypeStruct((NP, HID), jnp.float32),
            jax.ShapeDtypeStruct((NP, HID), jnp.float32),
            jax.ShapeDtypeStruct((2, NP, FT_IN), jnp.float32),
        ],
    )(x, wl, bl[None, :], wr, br[None, :])


def _epi_body(num_ref, den_ref, b_ref, wl_ref, bl_ref, wr_ref, br_ref,
              xlf_ref, xrf_ref, xlc_ref):
    d = jnp.maximum(den_ref[...][:, 0:1], 1e-30)
    h = jnp.concatenate([num_ref[0] / d, num_ref[1] / d], axis=1) + b_ref[...]
    h = h * 0.5 * (1.0 + lax.erf(h * 0.7071067811865476))
    l = jnp.dot(h, wl_ref[...], preferred_element_type=jnp.float32) + bl_ref[...]
    r = jnp.dot(h, wr_ref[...], preferred_element_type=jnp.float32) + br_ref[...]
    xlf_ref[...] = l
    xrf_ref[...] = r
    xlc_ref[0] = l[:, :FT_IN]
    xlc_ref[1] = l[:, FT_IN:]


def _epilogue_proj(num, den, b, wl, bl, wr, br):
    grid = NP // 1024
    return pl.pallas_call(
        _epi_body,
        grid=(grid,),
        in_specs=[
            pl.BlockSpec((2, 1024, FT_IN), lambda i: (0, i, 0)),
            pl.BlockSpec((1024, DW), lambda i: (i, 0)),
            pl.BlockSpec((1, HID), lambda i: (0, 0)),
            pl.BlockSpec((HID, HID), lambda i: (0, 0)),
            pl.BlockSpec((1, HID), lambda i: (0, 0)),
            pl.BlockSpec((HID, HID), lambda i: (0, 0)),
            pl.BlockSpec((1, HID), lambda i: (0, 0)),
        ],
        out_specs=[
            pl.BlockSpec((1024, HID), lambda i: (i, 0)),
            pl.BlockSpec((1024, HID), lambda i: (i, 0)),
            pl.BlockSpec((2, 1024, FT_IN), lambda i: (0, i, 0)),
        ],
        out_shape=[
            jax.ShapeDtypeStruct((NP, HID), jnp.float32),
            jax.ShapeDtypeStruct((NP, HID), jnp.float32),
            jax.ShapeDtypeStruct((2, NP, FT_IN), jnp.float32),
        ],
    )(num, den, b[None, :], wl, bl[None, :], wr, br[None, :])


def _pool_body(num_ref, den_ref, b_ref, batch_ref, wlin_ref, blin_ref, out_ref):
    d = jnp.maximum(den_ref[...][:, 0:1], 1e-30)
    h = jnp.concatenate([num_ref[0] / d, num_ref[1] / d], axis=1) + b_ref[...]
    gid = lax.broadcasted_iota(jnp.int32, (G, NP), 0)
    m = (gid == batch_ref[...]).astype(jnp.float32)
    cnt = jnp.sum(m, axis=1, keepdims=True)
    sums = jnp.dot(m, h, preferred_element_type=jnp.float32)
    pooled = sums / jnp.maximum(cnt, 1.0)
    out_ref[...] = (jnp.dot(pooled, wlin_ref[...],
                            preferred_element_type=jnp.float32) + blin_ref[...])


def _pool(num, den, b, batch_pad, wlin, blin):
    return pl.pallas_call(
        _pool_body,
        out_shape=jax.ShapeDtypeStruct((G, FT_IN), jnp.float32),
    )(num, den, b[None, :], batch_pad, wlin, blin[None, :])


def _phase1_body(srcp_hbm, dstp_hbm, att_hbm, xlf_hbm, xrf_hbm,
                 a_hbm, den_hbm,
                 sidx0, sidx1, didx0, didx1, lrow0, lrow1, rrow0, rrow1,
                 attv, av, denbuf, den_sp, sem0, sem1):
    sidx = (sidx0, sidx1)
    didx = (didx0, didx1)
    lrow = (lrow0, lrow1)
    rrow = (rrow0, rrow1)
    sem = (sem0, sem1)
    c = lax.axis_index("c")
    s = lax.axis_index("s")
    wid = s * 2 + c
    pltpu.sync_copy(att_hbm, attv)
    atts = [attv[k] for k in range(16)]
    lane = lax.iota(jnp.int32, 16)
    col0 = lane * 0
    zero16 = col0.astype(jnp.float32)
    for i in range(C1):
        denbuf[i] = zero16
    for j in range(RPW // C1):
        pltpu.sync_copy(denbuf, den_sp.at[pl.ds(s * RPW + j * C1, C1)])
    plsc.subcore_barrier()

    def load_idx(ci, p):
        off = wid * EW1 + ci * C1
        pltpu.sync_copy(srcp_hbm.at[0, pl.ds(off, C1)], sidx[p])
        pltpu.sync_copy(dstp_hbm.at[0, pl.ds(off, C1)], didx[p])

    def start_gather(p):
        pltpu.async_copy(xlf_hbm.at[sidx[p]], lrow[p], sem[p])
        pltpu.async_copy(xrf_hbm.at[didx[p]], rrow[p], sem[p])

    def wait_gather(p):
        pltpu.make_async_copy(xlf_hbm.at[sidx[p]], lrow[p], sem[p]).wait()
        pltpu.make_async_copy(xrf_hbm.at[didx[p]], rrow[p], sem[p]).wait()

    def compute(ci, p):
        off = wid * EW1 + ci * C1

        def group(g, carry2):
            def edge(e, res):
                row = g * 16 + e
                acc = zero16
                for k in range(16):
                    z = (lrow[p][row, pl.ds(k * 16, 16)]
                         + rrow[p][row, pl.ds(k * 16, 16)])
                    z = jnp.maximum(z, z * 0.2)
                    acc = acc + atts[k] * z
                return jnp.where(lane == e, _lanesum(acc, lane), res)

            res = lax.fori_loop(0, 16, edge, zero16)
            a16 = jnp.exp(res)
            av[pl.ds(g * 16, 16)] = a16
            for e in range(16):
                sc = _vshuf(a16, col0 + e)
                denbuf[g * 16 + e] = jnp.where(lane == 0, sc, zero16)
            return carry2

        lax.fori_loop(0, C1 // 16, group, 0)
        pltpu.sync_copy(av, a_hbm.at[pl.ds(off, C1)])
        pltpu.sync_copy(denbuf, den_sp.at[didx[p]], add=True)

    def chunk1(ci, carry):
        load_idx(ci, 0)
        start_gather(0)
        wait_gather(0)
        compute(ci, 0)
        return carry

    lax.fori_loop(0, NC1, chunk1, 0)
    plsc.subcore_barrier()
    for j in range(RPW // C1):
        sl = pl.ds(s * RPW + j * C1, C1)
        pltpu.sync_copy(den_sp.at[sl], denbuf)
        pltpu.sync_copy(denbuf, den_hbm.at[pl.ds(c * NP + s * RPW + j * C1, C1)])


def _phase1(srcp2, dstp2, attm, xlf, xrf):
    mesh = plsc.VectorSubcoreMesh(**_SC_MESH)
    f = functools.partial(
        pl.kernel,
        mesh=mesh,
        out_type=[
            jax.ShapeDtypeStruct((E_PAD,), jnp.float32),
            jax.ShapeDtypeStruct((2 * NP, DW), jnp.float32),
        ],
        scratch_types=[
            pltpu.VMEM((C1,), jnp.int32),
            pltpu.VMEM((C1,), jnp.int32),
            pltpu.VMEM((C1,), jnp.int32),
            pltpu.VMEM((C1,), jnp.int32),
            pltpu.VMEM((C1, HID), jnp.float32),
            pltpu.VMEM((C1, HID), jnp.float32),
            pltpu.VMEM((C1, HID), jnp.float32),
            pltpu.VMEM((C1, HID), jnp.float32),
            pltpu.VMEM((16, 16), jnp.float32),
            pltpu.VMEM((C1,), jnp.float32),
            pltpu.VMEM((C1, DW), jnp.float32),
            pltpu.VMEM_SHARED((NP, DW), jnp.float32),
            pltpu.SemaphoreType.DMA,
            pltpu.SemaphoreType.DMA,
        ],
    )(_phase1_body)
    return f(srcp2, dstp2, attm, xlf, xrf)


def _phase2_body(srcp_hbm, dstp_hbm, a_hbm, xlc_hbm,
                 num_hbm,
                 sidx0, sidx1, didx0, didx1, av0, av1, rows0, rows1,
                 acc_sp, sem0, sem1):
    sidx = (sidx0, sidx1)
    didx = (didx0, didx1)
    av = (av0, av1)
    rows = (rows0, rows1)
    sem = (sem0, sem1)
    c = lax.axis_index("c")
    s = lax.axis_index("s")
    lane = lax.iota(jnp.int32, 16)
    col0 = lane * 0
    zero16 = col0.astype(jnp.float32)
    for i in range(C2):
        for k in range(FT_IN // 16):
            rows0[i, pl.ds(k * 16, 16)] = zero16
    for j in range(RPW // C2):
        pltpu.sync_copy(rows0, acc_sp.at[pl.ds(s * RPW + j * C2, C2)])
    plsc.subcore_barrier()

    def load_idx(ci, p):
        off = s * EW2 + ci * C2
        pltpu.sync_copy(srcp_hbm.at[pl.ds(c * E_PAD + off, C2)], sidx[p])
        pltpu.sync_copy(dstp_hbm.at[0, pl.ds(off, C2)], didx[p])
        pltpu.sync_copy(a_hbm.at[pl.ds(off, C2)], av[p])

    def start_gather(p):
        pltpu.async_copy(xlc_hbm.at[sidx[p]], rows[p], sem[p])

    def wait_gather(p):
        pltpu.make_async_copy(xlc_hbm.at[sidx[p]], rows[p], sem[p]).wait()

    def compute(ci, p):
        for g in range(C2 // 16):
            a16 = av[p][pl.ds(g * 16, 16)]
            for e in range(16):
                row = g * 16 + e
                sc = _vshuf(a16, col0 + e)
                for k in range(FT_IN // 16):
                    sl = pl.ds(k * 16, 16)
                    rows[p][row, sl] = rows[p][row, sl] * sc
        pltpu.sync_copy(rows[p], acc_sp.at[didx[p]], add=True)

    if _DB2:
        load_idx(0, 0)
        start_gather(0)

        def chunk2(i, carry):
            for p in (0, 1):
                ci = 2 * i + p
                cn = jnp.minimum(ci + 1, NC2 - 1)
                load_idx(cn, 1 - p)
                start_gather(1 - p)
                wait_gather(p)
                compute(ci, p)
            return carry

        lax.fori_loop(0, NC2 // 2, chunk2, 0)
        wait_gather(0)
    else:
        def chunk1(ci, carry):
            load_idx(ci, 0)
            start_gather(0)
            wait_gather(0)
            compute(ci, 0)
            return carry

        lax.fori_loop(0, NC2, chunk1, 0)
    plsc.subcore_barrier()
    for j in range(RPW // C2):
        sl = pl.ds(s * RPW + j * C2, C2)
        pltpu.sync_copy(acc_sp.at[sl], rows0)
        pltpu.sync_copy(rows0, num_hbm.at[pl.ds(c * NP + s * RPW + j * C2, C2)])


def _phase2(srcp2, dstp2, a, xlc):
    mesh = plsc.VectorSubcoreMesh(**_SC_MESH)
    srcp_flat = srcp2.reshape(2 * E_PAD)
    f = functools.partial(
        pl.kernel,
        mesh=mesh,
        out_type=jax.ShapeDtypeStruct((2 * NP, FT_IN), jnp.float32),
        scratch_types=[
            pltpu.VMEM((C2,), jnp.int32),
            pltpu.VMEM((C2,), jnp.int32),
            pltpu.VMEM((C2,), jnp.int32),
            pltpu.VMEM((C2,), jnp.int32),
            pltpu.VMEM((C2,), jnp.float32),
            pltpu.VMEM((C2,), jnp.float32),
            pltpu.VMEM((C2, FT_IN), jnp.float32),
            pltpu.VMEM((C2 if _DB2 else 8, FT_IN), jnp.float32),
            pltpu.VMEM_SHARED((NP, FT_IN), jnp.float32),
            pltpu.SemaphoreType.DMA,
            pltpu.SemaphoreType.DMA,
        ],
    )(_phase2_body)
    return f(srcp_flat, dstp2, a, xlc)


def kernel(x, edge_index, batch,
           Wl1, bl1, Wr1, br1, att1, b1,
           Wl2, bl2, Wr2, br2, att2, b2,
           Wl3, bl3, Wr3, br3, att3, b3,
           Wlin, blin):
    i32 = jnp.int32
    loops = jnp.arange(N, dtype=i32)
    pad_e = jnp.full((E_PAD - E0,), N, dtype=i32)
    srcp = jnp.concatenate([edge_index[0].astype(i32), loops, pad_e])
    dstp = jnp.concatenate([edge_index[1].astype(i32), loops, pad_e])
    srcp2 = jnp.stack([srcp, srcp + NP])
    dstp2 = jnp.stack([dstp, dstp + NP])
    xpad = jnp.zeros((NP, FT_IN), jnp.float32).at[:N].set(x)
    batch_pad = jnp.full((1, NP), G, dtype=i32).at[0, :N].set(batch.astype(i32))

    params = [
        (Wl1, bl1, Wr1, br1, att1, b1),
        (Wl2, bl2, Wr2, br2, att2, b2),
        (Wl3, bl3, Wr3, br3, att3, b3),
    ]

    xlf, xrf, xlc = _proj(xpad, Wl1, bl1, Wr1, br1, FT_IN)
    for li in range(3):
        att = params[li][4].reshape(16, 16)
        a, den2 = _phase1(srcp2, dstp2, att, xlf, xrf)
        den = den2[:NP] + den2[NP:]
        xlc2 = xlc.reshape(2 * NP, FT_IN)
        numf = _phase2(srcp2, dstp2, a, xlc2)
        num = numf.reshape(2, NP, FT_IN)
        b = params[li][5]
        if li < 2:
            wl, bl, wr, br = params[li + 1][:4]
            xlf, xrf, xlc = _epilogue_proj(num, den, b, wl, bl, wr, br)
        else:
            out = _pool(num, den, b, batch_pad, Wlin, blin)
    return out


# speedup vs baseline: 4.9604x; 1.0011x over previous
import functools

import jax
import jax.numpy as jnp
from jax import lax
from jax.experimental import pallas as pl
from jax.experimental.pallas import tpu as pltpu
from jax.experimental.pallas import tpu_sc as plsc

N = 10000
G = 64
FT_IN = 128
HID = 256
NP = 10240
E0 = 330000
W1 = 32
C1 = 64
EW1 = 10368
NC1 = EW1 // C1
E_PAD = EW1 * W1
C2 = 64
EW2 = E_PAD // 16
NC2 = EW2 // C2
RPW = NP // 16
DW = 16

_SC_MESH = dict(core_axis_name="c", subcore_axis_name="s")

_GDN = lax.GatherDimensionNumbers(
    offset_dims=(), collapsed_slice_dims=(0,), start_index_map=(0,))


def _vshuf(v, idx):
    return lax.gather(v, idx[:, None], _GDN, (1,),
                      mode=lax.GatherScatterMode.PROMISE_IN_BOUNDS)


def _lanesum(v, lane):
    for sh in (8, 4, 2, 1):
        v = v + _vshuf(v, lane ^ sh)
    return v


def _proj_body(x_ref, wl_ref, bl_ref, wr_ref, br_ref,
               xlf_ref, xrf_ref, xlc_ref):
    xb = x_ref[...]
    l = jnp.dot(xb, wl_ref[...], preferred_element_type=jnp.float32) + bl_ref[...]
    r = jnp.dot(xb, wr_ref[...], preferred_element_type=jnp.float32) + br_ref[...]
    xlf_ref[...] = l
    xrf_ref[...] = r
    xlc_ref[0] = l[:, :FT_IN]
    xlc_ref[1] = l[:, FT_IN:]


def _proj(x, wl, bl, wr, br, din):
    grid = NP // 1024
    return pl.pallas_call(
        _proj_body,
        grid=(grid,),
        in_specs=[
            pl.BlockSpec((1024, din), lambda i: (i, 0)),
            pl.BlockSpec((din, HID), lambda i: (0, 0)),
            pl.BlockSpec((1, HID), lambda i: (0, 0)),
            pl.BlockSpec((din, HID), lambda i: (0, 0)),
            pl.BlockSpec((1, HID), lambda i: (0, 0)),
        ],
        out_specs=[
            pl.BlockSpec((1024, HID), lambda i: (i, 0)),
            pl.BlockSpec((1024, HID), lambda i: (i, 0)),
            pl.BlockSpec((2, 1024, FT_IN), lambda i: (0, i, 0)),
        ],
        out_shape=[
            jax.ShapeDtypeStruct((NP, HID), jnp.float32),
            jax.ShapeDtypeStruct((NP, HID), jnp.float32),
            jax.ShapeDtypeStruct((2, NP, FT_IN), jnp.float32),
        ],
    )(x, wl, bl[None, :], wr, br[None, :])


def _epi_body(num_ref, den_ref, b_ref, wl_ref, bl_ref, wr_ref, br_ref,
              xlf_ref, xrf_ref, xlc_ref):
    d = jnp.maximum(den_ref[...][:, 0:1], 1e-30)
    h = jnp.concatenate([num_ref[0] / d, num_ref[1] / d], axis=1) + b_ref[...]
    h = h * 0.5 * (1.0 + lax.erf(h * 0.7071067811865476))
    l = jnp.dot(h, wl_ref[...], preferred_element_type=jnp.float32) + bl_ref[...]
    r = jnp.dot(h, wr_ref[...], preferred_element_type=jnp.float32) + br_ref[...]
    xlf_ref[...] = l
    xrf_ref[...] = r
    xlc_ref[0] = l[:, :FT_IN]
    xlc_ref[1] = l[:, FT_IN:]


def _epilogue_proj(num, den, b, wl, bl, wr, br):
    grid = NP // 1024
    return pl.pallas_call(
        _epi_body,
        grid=(grid,),
        in_specs=[
            pl.BlockSpec((2, 1024, FT_IN), lambda i: (0, i, 0)),
            pl.BlockSpec((1024, DW), lambda i: (i, 0)),
            pl.BlockSpec((1, HID), lambda i: (0, 0)),
            pl.BlockSpec((HID, HID), lambda i: (0, 0)),
            pl.BlockSpec((1, HID), lambda i: (0, 0)),
            pl.BlockSpec((HID, HID), lambda i: (0, 0)),
            pl.BlockSpec((1, HID), lambda i: (0, 0)),
        ],
        out_specs=[
            pl.BlockSpec((1024, HID), lambda i: (i, 0)),
            pl.BlockSpec((1024, HID), lambda i: (i, 0)),
            pl.BlockSpec((2, 1024, FT_IN), lambda i: (0, i, 0)),
        ],
        out_shape=[
            jax.ShapeDtypeStruct((NP, HID), jnp.float32),
            jax.ShapeDtypeStruct((NP, HID), jnp.float32),
            jax.ShapeDtypeStruct((2, NP, FT_IN), jnp.float32),
        ],
    )(num, den, b[None, :], wl, bl[None, :], wr, br[None, :])


def _pool_body(num_ref, den_ref, b_ref, batch_ref, wlin_ref, blin_ref, out_ref):
    d = jnp.maximum(den_ref[...][:, 0:1], 1e-30)
    h = jnp.concatenate([num_ref[0] / d, num_ref[1] / d], axis=1) + b_ref[...]
    gid = lax.broadcasted_iota(jnp.int32, (G, NP), 0)
    m = (gid == batch_ref[...]).astype(jnp.float32)
    cnt = jnp.sum(m, axis=1, keepdims=True)
    sums = jnp.dot(m, h, preferred_element_type=jnp.float32)
    pooled = sums / jnp.maximum(cnt, 1.0)
    out_ref[...] = (jnp.dot(pooled, wlin_ref[...],
                            preferred_element_type=jnp.float32) + blin_ref[...])


def _pool(num, den, b, batch_pad, wlin, blin):
    return pl.pallas_call(
        _pool_body,
        out_shape=jax.ShapeDtypeStruct((G, FT_IN), jnp.float32),
    )(num, den, b[None, :], batch_pad, wlin, blin[None, :])


def _phase1_body(srcp_hbm, dstp_hbm, att_hbm, xlf_hbm, xrf_hbm,
                 a_hbm, den_hbm,
                 sidx0, sidx1, didx0, didx1, lrow0, lrow1, rrow0, rrow1,
                 attv, av, denbuf, den_sp, sem0, sem1):
    sidx = (sidx0, sidx1)
    didx = (didx0, didx1)
    lrow = (lrow0, lrow1)
    rrow = (rrow0, rrow1)
    sem = (sem0, sem1)
    c = lax.axis_index("c")
    s = lax.axis_index("s")
    wid = s * 2 + c
    pltpu.sync_copy(att_hbm, attv)
    atts = [attv[k] for k in range(16)]
    lane = lax.iota(jnp.int32, 16)
    col0 = lane * 0
    zero16 = col0.astype(jnp.float32)
    for i in range(C1):
        denbuf[i] = zero16
    for j in range(RPW // C1):
        pltpu.sync_copy(denbuf, den_sp.at[pl.ds(s * RPW + j * C1, C1)])
    plsc.subcore_barrier()

    def load_idx(ci, p):
        off = wid * EW1 + ci * C1
        pltpu.sync_copy(srcp_hbm.at[0, pl.ds(off, C1)], sidx[p])
        pltpu.sync_copy(dstp_hbm.at[0, pl.ds(off, C1)], didx[p])

    def start_gather(p):
        pltpu.async_copy(xlf_hbm.at[sidx[p]], lrow[p], sem[p])
        pltpu.async_copy(xrf_hbm.at[didx[p]], rrow[p], sem[p])

    def wait_gather(p):
        pltpu.make_async_copy(xlf_hbm.at[sidx[p]], lrow[p], sem[p]).wait()
        pltpu.make_async_copy(xrf_hbm.at[didx[p]], rrow[p], sem[p]).wait()

    def compute(ci, p):
        off = wid * EW1 + ci * C1

        def group(g, carry2):
            def edge(e, res):
                row = g * 16 + e
                acc = zero16
                for k in range(16):
                    z = (lrow[p][row, pl.ds(k * 16, 16)]
                         + rrow[p][row, pl.ds(k * 16, 16)])
                    z = jnp.maximum(z, z * 0.2)
                    acc = acc + atts[k] * z
                return jnp.where(lane == e, _lanesum(acc, lane), res)

            res = lax.fori_loop(0, 16, edge, zero16)
            a16 = jnp.exp(res)
            av[pl.ds(g * 16, 16)] = a16
            for e in range(16):
                sc = _vshuf(a16, col0 + e)
                denbuf[g * 16 + e] = jnp.where(lane == 0, sc, zero16)
            return carry2

        lax.fori_loop(0, C1 // 16, group, 0)
        pltpu.sync_copy(av, a_hbm.at[pl.ds(off, C1)])
        pltpu.sync_copy(denbuf, den_sp.at[didx[p]], add=True)

    def chunk1(ci, carry):
        load_idx(ci, 0)
        start_gather(0)
        wait_gather(0)
        compute(ci, 0)
        return carry

    lax.fori_loop(0, NC1, chunk1, 0)
    plsc.subcore_barrier()
    for j in range(RPW // C1):
        sl = pl.ds(s * RPW + j * C1, C1)
        pltpu.sync_copy(den_sp.at[sl], denbuf)
        pltpu.sync_copy(denbuf, den_hbm.at[pl.ds(c * NP + s * RPW + j * C1, C1)])


def _phase1(srcp2, dstp2, attm, xlf, xrf):
    mesh = plsc.VectorSubcoreMesh(**_SC_MESH)
    f = functools.partial(
        pl.kernel,
        mesh=mesh,
        out_type=[
            jax.ShapeDtypeStruct((E_PAD,), jnp.float32),
            jax.ShapeDtypeStruct((2 * NP, DW), jnp.float32),
        ],
        scratch_types=[
            pltpu.VMEM((C1,), jnp.int32),
            pltpu.VMEM((C1,), jnp.int32),
            pltpu.VMEM((C1,), jnp.int32),
            pltpu.VMEM((C1,), jnp.int32),
            pltpu.VMEM((C1, HID), jnp.float32),
            pltpu.VMEM((C1, HID), jnp.float32),
            pltpu.VMEM((C1, HID), jnp.float32),
            pltpu.VMEM((C1, HID), jnp.float32),
            pltpu.VMEM((16, 16), jnp.float32),
            pltpu.VMEM((C1,), jnp.float32),
            pltpu.VMEM((C1, DW), jnp.float32),
            pltpu.VMEM_SHARED((NP, DW), jnp.float32),
            pltpu.SemaphoreType.DMA,
            pltpu.SemaphoreType.DMA,
        ],
    )(_phase1_body)
    return f(srcp2, dstp2, attm, xlf, xrf)


def _phase2_body(srcp_hbm, dstp_hbm, a_hbm, xlc_hbm,
                 num_hbm,
                 sidx0, sidx1, didx0, didx1, av0, av1, rows0, rows1,
                 acc_sp, sem0, sem1):
    sidx = (sidx0, sidx1)
    didx = (didx0, didx1)
    av = (av0, av1)
    rows = (rows0, rows1)
    sem = (sem0, sem1)
    c = lax.axis_index("c")
    s = lax.axis_index("s")
    lane = lax.iota(jnp.int32, 16)
    col0 = lane * 0
    zero16 = col0.astype(jnp.float32)
    for i in range(C2):
        for k in range(FT_IN // 16):
            rows0[i, pl.ds(k * 16, 16)] = zero16
    for j in range(RPW // C2):
        pltpu.sync_copy(rows0, acc_sp.at[pl.ds(s * RPW + j * C2, C2)])
    plsc.subcore_barrier()

    def load_idx(ci, p):
        off = s * EW2 + ci * C2
        pltpu.sync_copy(srcp_hbm.at[pl.ds(c * E_PAD + off, C2)], sidx[p])
        pltpu.sync_copy(dstp_hbm.at[0, pl.ds(off, C2)], didx[p])
        pltpu.sync_copy(a_hbm.at[pl.ds(off, C2)], av[p])

    def start_gather(p):
        pltpu.async_copy(xlc_hbm.at[sidx[p]], rows[p], sem[p])

    def wait_gather(p):
        pltpu.make_async_copy(xlc_hbm.at[sidx[p]], rows[p], sem[p]).wait()

    def compute(ci, p):
        for g in range(C2 // 16):
            a16 = av[p][pl.ds(g * 16, 16)]
            for e in range(16):
                row = g * 16 + e
                sc = _vshuf(a16, col0 + e)
                for k in range(FT_IN // 16):
                    sl = pl.ds(k * 16, 16)
                    rows[p][row, sl] = rows[p][row, sl] * sc
        pltpu.sync_copy(rows[p], acc_sp.at[didx[p]], add=True)

    def chunk1(ci, carry):
        load_idx(ci, 0)
        start_gather(0)
        wait_gather(0)
        compute(ci, 0)
        return carry

    lax.fori_loop(0, NC2, chunk1, 0)
    plsc.subcore_barrier()
    for j in range(RPW // C2):
        sl = pl.ds(s * RPW + j * C2, C2)
        pltpu.sync_copy(acc_sp.at[sl], rows0)
        pltpu.sync_copy(rows0, num_hbm.at[pl.ds(c * NP + s * RPW + j * C2, C2)])


def _phase2(srcp2, dstp2, a, xlc):
    mesh = plsc.VectorSubcoreMesh(**_SC_MESH)
    srcp_flat = srcp2.reshape(2 * E_PAD)
    f = functools.partial(
        pl.kernel,
        mesh=mesh,
        out_type=jax.ShapeDtypeStruct((2 * NP, FT_IN), jnp.float32),
        scratch_types=[
            pltpu.VMEM((C2,), jnp.int32),
            pltpu.VMEM((C2,), jnp.int32),
            pltpu.VMEM((C2,), jnp.int32),
            pltpu.VMEM((C2,), jnp.int32),
            pltpu.VMEM((C2,), jnp.float32),
            pltpu.VMEM((C2,), jnp.float32),
            pltpu.VMEM((C2, FT_IN), jnp.float32),
            pltpu.VMEM((8, FT_IN), jnp.float32),
            pltpu.VMEM_SHARED((NP, FT_IN), jnp.float32),
            pltpu.SemaphoreType.DMA,
            pltpu.SemaphoreType.DMA,
        ],
    )(_phase2_body)
    return f(srcp_flat, dstp2, a, xlc)


def kernel(x, edge_index, batch,
           Wl1, bl1, Wr1, br1, att1, b1,
           Wl2, bl2, Wr2, br2, att2, b2,
           Wl3, bl3, Wr3, br3, att3, b3,
           Wlin, blin):
    i32 = jnp.int32
    loops = jnp.arange(N, dtype=i32)
    pad_e = jnp.full((E_PAD - E0,), N, dtype=i32)
    srcp = jnp.concatenate([edge_index[0].astype(i32), loops, pad_e])
    dstp = jnp.concatenate([edge_index[1].astype(i32), loops, pad_e])
    srcp2 = jnp.stack([srcp, srcp + NP])
    dstp2 = jnp.stack([dstp, dstp + NP])
    xpad = jnp.zeros((NP, FT_IN), jnp.float32).at[:N].set(x)
    batch_pad = jnp.full((1, NP), G, dtype=i32).at[0, :N].set(batch.astype(i32))

    params = [
        (Wl1, bl1, Wr1, br1, att1, b1),
        (Wl2, bl2, Wr2, br2, att2, b2),
        (Wl3, bl3, Wr3, br3, att3, b3),
    ]

    xlf, xrf, xlc = _proj(xpad, Wl1, bl1, Wr1, br1, FT_IN)
    for li in range(3):
        att = params[li][4].reshape(16, 16)
        a, den2 = _phase1(srcp2, dstp2, att, xlf, xrf)
        den = den2[:NP] + den2[NP:]
        xlc2 = xlc.reshape(2 * NP, FT_IN)
        numf = _phase2(srcp2, dstp2, a, xlc2)
        num = numf.reshape(2, NP, FT_IN)
        b = params[li][5]
        if li < 2:
            wl, bl, wr, br = params[li + 1][:4]
            xlf, xrf, xlc = _epilogue_proj(num, den, b, wl, bl, wr, br)
        else:
            out = _pool(num, den, b, batch_pad, Wlin, blin)
    return out
